# baseline (speedup 1.0000x reference)
import jax
import jax.numpy as jnp
from jax import lax
from jax.experimental import pallas as pl
from jax.experimental.pallas import tpu as pltpu
from jax.experimental.pallas import tpu_sc as plsc

_NUM_CORES = 2
_NW = 32

_TEC_ROWS = 6144
_TC_CHUNK = 16
_TC_NBUF = 4
_TC_LA = 2
_SC_CHUNK = 256
_SC_NBUF = 3
_SC_LA = 2


def _ring_copy(w_hbm, out_hbm, buf, gsems, ssems, base, total_rows, chunk,
               nbuf, la):
    nchunks = total_rows // chunk

    def gather(j):
        k = j % nbuf
        return pltpu.async_copy(
            w_hbm.at[pl.ds(base + j * chunk, chunk)], buf.at[k], gsems[k]
        )

    def scatter(i):
        k = i % nbuf
        return pltpu.async_copy(
            buf.at[k], out_hbm.at[pl.ds(base + i * chunk, chunk)], ssems[k]
        )

    gd, sd = {}, {}
    for j in range(min(la, nchunks)):
        gd[j] = gather(j)
    for i in range(nchunks):
        gd[i].wait()
        sd[i] = scatter(i)
        j = i + la
        if j < nchunks:
            if j >= nbuf:
                sd[j - nbuf].wait()
            gd[j] = gather(j)
    for i in range(max(0, nchunks - nbuf), nchunks):
        sd[i].wait()


def _tec_fn(w_hbm, out_hbm, tbuf, sbuf, *sems):
    gsems = sems[:_TC_NBUF]
    ssems = sems[_TC_NBUF:2 * _TC_NBUF]
    rpw = _TEC_ROWS // _NW
    wid = lax.axis_index("s") * _NUM_CORES + lax.axis_index("c")
    _ring_copy(w_hbm, out_hbm, tbuf, gsems, ssems, wid * rpw, rpw,
               _TC_CHUNK, _TC_NBUF, _TC_LA)


def _scs_fn(w_hbm, out_hbm, tbuf, sbuf, *sems):
    gsems = sems[2 * _TC_NBUF:2 * _TC_NBUF + _SC_NBUF]
    ssems = sems[2 * _TC_NBUF + _SC_NBUF:]
    rows = out_hbm.shape[0]
    rpc = (rows - _TEC_ROWS) // _NUM_CORES
    cid = lax.axis_index("c")
    _ring_copy(w_hbm, out_hbm, sbuf, gsems, ssems, _TEC_ROWS + cid * rpc,
               rpc, _SC_CHUNK, _SC_NBUF, _SC_LA)


def kernel(x, W):
    seq_len = x.shape[1]
    dim = W.shape[1]
    tec_mesh = plsc.VectorSubcoreMesh(core_axis_name="c", subcore_axis_name="s")
    scs_mesh = plsc.ScalarSubcoreMesh(axis_name="c")
    k = pl.kernel(
        [_tec_fn, _scs_fn],
        out_type=jax.ShapeDtypeStruct((seq_len, dim), W.dtype),
        mesh=[tec_mesh, scs_mesh],
        scratch_types=(
            [(pltpu.VMEM @ tec_mesh)((_TC_NBUF, _TC_CHUNK, dim), jnp.float32),
             pltpu.VMEM_SHARED((_SC_NBUF, _SC_CHUNK, dim), jnp.float32)]
            + [pltpu.SemaphoreType.DMA @ tec_mesh] * (2 * _TC_NBUF)
            + [pltpu.SemaphoreType.DMA @ scs_mesh] * (2 * _SC_NBUF)
        ),
    )
    return k(W)


# speedup vs baseline: 1.0025x; 1.0025x over previous
import jax
import jax.numpy as jnp
from jax import lax
from jax.experimental import pallas as pl
from jax.experimental.pallas import tpu as pltpu
from jax.experimental.pallas import tpu_sc as plsc

_NUM_CORES = 2
_NW = 32

_TEC_ROWS = 5632
_TC_CHUNK = 16
_TC_NBUF = 4
_TC_LA = 2
_SC_CHUNK = 256
_SC_NBUF = 3
_SC_LA = 2


def _ring_copy(w_hbm, out_hbm, buf, gsems, ssems, base, total_rows, chunk,
               nbuf, la):
    nchunks = total_rows // chunk

    def gather(j):
        k = j % nbuf
        return pltpu.async_copy(
            w_hbm.at[pl.ds(base + j * chunk, chunk)], buf.at[k], gsems[k]
        )

    def scatter(i):
        k = i % nbuf
        return pltpu.async_copy(
            buf.at[k], out_hbm.at[pl.ds(base + i * chunk, chunk)], ssems[k]
        )

    gd, sd = {}, {}
    for j in range(min(la, nchunks)):
        gd[j] = gather(j)
    for i in range(nchunks):
        gd[i].wait()
        sd[i] = scatter(i)
        j = i + la
        if j < nchunks:
            if j >= nbuf:
                sd[j - nbuf].wait()
            gd[j] = gather(j)
    for i in range(max(0, nchunks - nbuf), nchunks):
        sd[i].wait()


def _tec_fn(w_hbm, out_hbm, tbuf, sbuf, *sems):
    gsems = sems[:_TC_NBUF]
    ssems = sems[_TC_NBUF:2 * _TC_NBUF]
    rpw = _TEC_ROWS // _NW
    wid = lax.axis_index("s") * _NUM_CORES + lax.axis_index("c")
    _ring_copy(w_hbm, out_hbm, tbuf, gsems, ssems, wid * rpw, rpw,
               _TC_CHUNK, _TC_NBUF, _TC_LA)


def _scs_fn(w_hbm, out_hbm, tbuf, sbuf, *sems):
    gsems = sems[2 * _TC_NBUF:2 * _TC_NBUF + _SC_NBUF]
    ssems = sems[2 * _TC_NBUF + _SC_NBUF:]
    rows = out_hbm.shape[0]
    rpc = (rows - _TEC_ROWS) // _NUM_CORES
    cid = lax.axis_index("c")
    _ring_copy(w_hbm, out_hbm, sbuf, gsems, ssems, _TEC_ROWS + cid * rpc,
               rpc, _SC_CHUNK, _SC_NBUF, _SC_LA)


def kernel(x, W):
    seq_len = x.shape[1]
    dim = W.shape[1]
    tec_mesh = plsc.VectorSubcoreMesh(core_axis_name="c", subcore_axis_name="s")
    scs_mesh = plsc.ScalarSubcoreMesh(axis_name="c")
    k = pl.kernel(
        [_tec_fn, _scs_fn],
        out_type=jax.ShapeDtypeStruct((seq_len, dim), W.dtype),
        mesh=[tec_mesh, scs_mesh],
        scratch_types=(
            [(pltpu.VMEM @ tec_mesh)((_TC_NBUF, _TC_CHUNK, dim), jnp.float32),
             pltpu.VMEM_SHARED((_SC_NBUF, _SC_CHUNK, dim), jnp.float32)]
            + [pltpu.SemaphoreType.DMA @ tec_mesh] * (2 * _TC_NBUF)
            + [pltpu.SemaphoreType.DMA @ scs_mesh] * (2 * _SC_NBUF)
        ),
    )
    return k(W)


# speedup vs baseline: 1.0131x; 1.0105x over previous
import jax
import jax.numpy as jnp
from jax import lax
from jax.experimental import pallas as pl
from jax.experimental.pallas import tpu as pltpu
from jax.experimental.pallas import tpu_sc as plsc

_NUM_CORES = 2
_NW = 32

_TEC_ROWS = 5120
_TC_CHUNK = 16
_TC_NBUF = 5
_TC_LA = 3
_SC_CHUNK = 128
_SC_NBUF = 3
_SC_LA = 2


def _ring_copy(w_hbm, out_hbm, buf, gsems, ssems, base, total_rows, chunk,
               nbuf, la):
    nchunks = total_rows // chunk

    def gather(j):
        k = j % nbuf
        return pltpu.async_copy(
            w_hbm.at[pl.ds(base + j * chunk, chunk)], buf.at[k], gsems[k]
        )

    def scatter(i):
        k = i % nbuf
        return pltpu.async_copy(
            buf.at[k], out_hbm.at[pl.ds(base + i * chunk, chunk)], ssems[k]
        )

    gd, sd = {}, {}
    for j in range(min(la, nchunks)):
        gd[j] = gather(j)
    for i in range(nchunks):
        gd[i].wait()
        sd[i] = scatter(i)
        j = i + la
        if j < nchunks:
            if j >= nbuf:
                sd[j - nbuf].wait()
            gd[j] = gather(j)
    for i in range(max(0, nchunks - nbuf), nchunks):
        sd[i].wait()


def _tec_fn(w_hbm, out_hbm, tbuf, sbuf, *sems):
    gsems = sems[:_TC_NBUF]
    ssems = sems[_TC_NBUF:2 * _TC_NBUF]
    rpw = _TEC_ROWS // _NW
    wid = lax.axis_index("s") * _NUM_CORES + lax.axis_index("c")
    _ring_copy(w_hbm, out_hbm, tbuf, gsems, ssems, wid * rpw, rpw,
               _TC_CHUNK, _TC_NBUF, _TC_LA)


def _scs_fn(w_hbm, out_hbm, tbuf, sbuf, *sems):
    gsems = sems[2 * _TC_NBUF:2 * _TC_NBUF + _SC_NBUF]
    ssems = sems[2 * _TC_NBUF + _SC_NBUF:]
    rows = out_hbm.shape[0]
    rpc = (rows - _TEC_ROWS) // _NUM_CORES
    cid = lax.axis_index("c")
    _ring_copy(w_hbm, out_hbm, sbuf, gsems, ssems, _TEC_ROWS + cid * rpc,
               rpc, _SC_CHUNK, _SC_NBUF, _SC_LA)


def kernel(x, W):
    seq_len = x.shape[1]
    dim = W.shape[1]
    tec_mesh = plsc.VectorSubcoreMesh(core_axis_name="c", subcore_axis_name="s")
    scs_mesh = plsc.ScalarSubcoreMesh(axis_name="c")
    k = pl.kernel(
        [_tec_fn, _scs_fn],
        out_type=jax.ShapeDtypeStruct((seq_len, dim), W.dtype),
        mesh=[tec_mesh, scs_mesh],
        scratch_types=(
            [(pltpu.VMEM @ tec_mesh)((_TC_NBUF, _TC_CHUNK, dim), jnp.float32),
             pltpu.VMEM_SHARED((_SC_NBUF, _SC_CHUNK, dim), jnp.float32)]
            + [pltpu.SemaphoreType.DMA @ tec_mesh] * (2 * _TC_NBUF)
            + [pltpu.SemaphoreType.DMA @ scs_mesh] * (2 * _SC_NBUF)
        ),
    )
    return k(W)


# speedup vs baseline: 1.0169x; 1.0037x over previous
import jax
import jax.numpy as jnp
from jax import lax
from jax.experimental import pallas as pl
from jax.experimental.pallas import tpu as pltpu
from jax.experimental.pallas import tpu_sc as plsc

_NUM_CORES = 2
_NW = 32

_TEC_ROWS = 5120
_TC_CHUNK = 16
_TC_NBUF = 4
_TC_LA = 2
_SC_CHUNK = 256
_SC_NBUF = 3
_SC_LA = 2


def _ring_copy(w_hbm, out_hbm, buf, gsems, ssems, base, total_rows, chunk,
               nbuf, la):
    nchunks = total_rows // chunk

    def gather(j):
        k = j % nbuf
        return pltpu.async_copy(
            w_hbm.at[pl.ds(base + j * chunk, chunk)], buf.at[k], gsems[k]
        )

    def scatter(i):
        k = i % nbuf
        return pltpu.async_copy(
            buf.at[k], out_hbm.at[pl.ds(base + i * chunk, chunk)], ssems[k]
        )

    gd, sd = {}, {}
    for j in range(min(la, nchunks)):
        gd[j] = gather(j)
    for i in range(nchunks):
        gd[i].wait()
        sd[i] = scatter(i)
        j = i + la
        if j < nchunks:
            if j >= nbuf:
                sd[j - nbuf].wait()
            gd[j] = gather(j)
    for i in range(max(0, nchunks - nbuf), nchunks):
        sd[i].wait()


def _tec_fn(w_hbm, out_hbm, tbuf, sbuf, *sems):
    gsems = sems[:_TC_NBUF]
    ssems = sems[_TC_NBUF:2 * _TC_NBUF]
    rpw = _TEC_ROWS // _NW
    wid = lax.axis_index("s") * _NUM_CORES + lax.axis_index("c")
    _ring_copy(w_hbm, out_hbm, tbuf, gsems, ssems, wid * rpw, rpw,
               _TC_CHUNK, _TC_NBUF, _TC_LA)


def _scs_fn(w_hbm, out_hbm, tbuf, sbuf, *sems):
    gsems = sems[2 * _TC_NBUF:2 * _TC_NBUF + _SC_NBUF]
    ssems = sems[2 * _TC_NBUF + _SC_NBUF:]
    rows = out_hbm.shape[0]
    rpc = (rows - _TEC_ROWS) // _NUM_CORES
    cid = lax.axis_index("c")
    _ring_copy(w_hbm, out_hbm, sbuf, gsems, ssems, _TEC_ROWS + cid * rpc,
               rpc, _SC_CHUNK, _SC_NBUF, _SC_LA)


def kernel(x, W):
    seq_len = x.shape[1]
    dim = W.shape[1]
    tec_mesh = plsc.VectorSubcoreMesh(core_axis_name="c", subcore_axis_name="s")
    scs_mesh = plsc.ScalarSubcoreMesh(axis_name="c")
    k = pl.kernel(
        [_tec_fn, _scs_fn],
        out_type=jax.ShapeDtypeStruct((seq_len, dim), W.dtype),
        mesh=[tec_mesh, scs_mesh],
        scratch_types=(
            [(pltpu.VMEM @ tec_mesh)((_TC_NBUF, _TC_CHUNK, dim), jnp.float32),
             pltpu.VMEM_SHARED((_SC_NBUF, _SC_CHUNK, dim), jnp.float32)]
            + [pltpu.SemaphoreType.DMA @ tec_mesh] * (2 * _TC_NBUF)
            + [pltpu.SemaphoreType.DMA @ scs_mesh] * (2 * _SC_NBUF)
        ),
    )
    return k(W)
